# baseline (speedup 1.0000x reference)
import functools

import jax
import jax.numpy as jnp
from jax import lax
from jax.experimental import pallas as pl
from jax.experimental.pallas import tpu as pltpu
from jax.experimental.pallas import tpu_sc as plsc

_NC = 2
_NS = 16
_NW = _NC * _NS
_C = 64


def _sc_body(tok_hbm, px_hbm, py_hbm, tabx_hbm, taby_hbm, out_hbm,
             pxc, pyc, xi, yi, tokb, xr, yr, semx, semy, semt):
    R = tabx_hbm.shape[0]
    rows_w = tok_hbm.shape[0] // _NW
    nch = rows_w // _C
    Dq = tabx_hbm.shape[1]
    wid = lax.axis_index("s") * _NC + lax.axis_index("c")
    base = wid * rows_w
    for c in range(nch):
        off = base + c * _C
        cp_t = pltpu.make_async_copy(
            tok_hbm.at[pl.ds(off, _C), pl.ds(0, 2 * Dq)], tokb, semt)
        cp_t.start()
        pltpu.sync_copy(px_hbm.at[pl.ds(off, _C)], pxc)
        pltpu.sync_copy(py_hbm.at[pl.ds(off, _C)], pyc)
        for j in range(_C // 16):
            s = pl.ds(j * 16, 16)
            xi[s] = (pxc[s] * float(R)).astype(jnp.int32)
            yi[s] = (pyc[s] * float(R)).astype(jnp.int32)
        cpx = pltpu.make_async_copy(tabx_hbm.at[xi], xr, semx)
        cpx.start()
        cpy = pltpu.make_async_copy(taby_hbm.at[yi], yr, semy)
        cpy.start()
        cp_t.wait()
        cpx.wait()
        cpy.wait()

        def addrow(r, carry):
            for j in range(Dq // 16):
                s = pl.ds(j * 16, 16)
                s2 = pl.ds(Dq + j * 16, 16)
                tokb[r, s] = tokb[r, s] + xr[r, s]
                tokb[r, s2] = tokb[r, s2] + yr[r, s]
            return carry

        lax.fori_loop(0, _C, addrow, 0)
        pltpu.sync_copy(tokb, out_hbm.at[pl.ds(off, _C)])


def _sc_spatial(tok, px, py, tabx, taby):
    BN, D = tok.shape
    Dq = tabx.shape[1]
    mesh = plsc.VectorSubcoreMesh(core_axis_name="c", subcore_axis_name="s",
                                  num_cores=_NC, num_subcores=_NS)
    f = pl.kernel(
        _sc_body,
        out_type=jax.ShapeDtypeStruct((BN, 2 * Dq), jnp.float32),
        mesh=mesh,
        scratch_types=[
            pltpu.VMEM((_C,), jnp.float32),
            pltpu.VMEM((_C,), jnp.float32),
            pltpu.VMEM((_C,), jnp.int32),
            pltpu.VMEM((_C,), jnp.int32),
            pltpu.VMEM((_C, 2 * Dq), jnp.float32),
            pltpu.VMEM((_C, Dq), jnp.float32),
            pltpu.VMEM((_C, Dq), jnp.float32),
            pltpu.SemaphoreType.DMA,
            pltpu.SemaphoreType.DMA,
            pltpu.SemaphoreType.DMA,
        ],
    )
    return f(tok, px, py, tabx, taby)


def _tc_body(tok_ref, tp_ref, w1_ref, b1_ref, w2_ref, b2_ref, out_ref):
    t = tp_ref[...]
    h = t * w1_ref[...] + b1_ref[...]
    h = 0.5 * h * (1.0 + jax.lax.erf(h * 0.7071067811865476))
    temp = jnp.dot(h.astype(jnp.bfloat16), w2_ref[...].astype(jnp.bfloat16),
                   preferred_element_type=jnp.float32) + b2_ref[...]
    out_ref[...] = tok_ref[...] + temp


def _tc_temporal(tok, tp, W1, b1r, W2, b2r):
    BN, D = tok.shape
    H = W1.shape[1]
    RB = 1024
    grid = (BN // RB,)
    rep = lambda i: (0, 0)
    return pl.pallas_call(
        _tc_body,
        grid=grid,
        in_specs=[
            pl.BlockSpec((RB, H), lambda i: (i, 1)),
            pl.BlockSpec((RB, 1), lambda i: (i, 0)),
            pl.BlockSpec((1, H), rep),
            pl.BlockSpec((1, H), rep),
            pl.BlockSpec((H, H), rep),
            pl.BlockSpec((1, H), rep),
        ],
        out_specs=pl.BlockSpec((RB, H), lambda i: (i, 0)),
        out_shape=jax.ShapeDtypeStruct((BN, H), jnp.float32),
        compiler_params=pltpu.CompilerParams(
            dimension_semantics=("arbitrary",),
        ),
    )(tok, tp, W1, b1r, W2, b2r)


@jax.jit
def kernel(tokens, spatial_positions, temporal_positions, spatial_embed_x,
           spatial_embed_y, W1, b1, W2, b2):
    B, N, D = tokens.shape
    BN = B * N
    R = spatial_embed_x.shape[1]
    H = W1.shape[1]

    tok = tokens.reshape(BN, D)
    sp = spatial_positions.reshape(BN, 2)
    px = sp[:, 0]
    py = sp[:, 1]
    tp = temporal_positions.reshape(BN, 1)
    tabx = spatial_embed_x.reshape(R, D // 4)
    taby = spatial_embed_y.reshape(R, D // 4)

    out_sp = _sc_spatial(tok, px, py, tabx, taby)
    out_tm = _tc_temporal(tok, tp, W1, b1.reshape(1, H), W2, b2.reshape(1, H))
    return jnp.concatenate([out_sp, out_tm], axis=1).reshape(B, N, D)


# speedup vs baseline: 1.2107x; 1.2107x over previous
import functools

import jax
import jax.numpy as jnp
from jax import lax
from jax.experimental import pallas as pl
from jax.experimental.pallas import tpu as pltpu
from jax.experimental.pallas import tpu_sc as plsc

_NC = 2
_NS = 16
_NW = _NC * _NS
_C = 64


def _sc_body(tok_hbm, px_hbm, py_hbm, tabx_hbm, taby_hbm, out_hbm,
             pxv, pyv, xi, yi, tok0, tok1, xr, yr,
             semi0, semi1, semo0, semo1, semg):
    R = tabx_hbm.shape[0]
    rows_w = tok_hbm.shape[0] // _NW
    nch = rows_w // _C
    Dq = tabx_hbm.shape[1]
    wid = lax.axis_index("s") * _NC + lax.axis_index("c")
    base = wid * rows_w

    pltpu.sync_copy(px_hbm.at[pl.ds(base, rows_w)], pxv)
    pltpu.sync_copy(py_hbm.at[pl.ds(base, rows_w)], pyv)
    for j in range(rows_w // 16):
        s = pl.ds(j * 16, 16)
        xi[s] = (pxv[s] * float(R)).astype(jnp.int32)
        yi[s] = (pyv[s] * float(R)).astype(jnp.int32)

    toks = (tok0, tok1)
    semis = (semi0, semi1)
    semos = (semo0, semo1)

    def start_tok_in(c):
        pltpu.make_async_copy(
            tok_hbm.at[pl.ds(base + c * _C, _C), pl.ds(0, 2 * Dq)],
            toks[c % 2], semis[c % 2]).start()

    def start_gathers(c):
        pltpu.make_async_copy(
            tabx_hbm.at[xi.at[pl.ds(c * _C, _C)]], xr, semg).start()
        pltpu.make_async_copy(
            taby_hbm.at[yi.at[pl.ds(c * _C, _C)]], yr, semg).start()

    start_tok_in(0)
    start_gathers(0)
    for c in range(nch):
        p = c % 2
        off = base + c * _C
        tokb = toks[p]
        pltpu.make_async_copy(
            tok_hbm.at[pl.ds(off, _C), pl.ds(0, 2 * Dq)], tokb,
            semis[p]).wait()
        pltpu.make_async_copy(tabx_hbm.at[xi.at[pl.ds(c * _C, _C)]], xr,
                              semg).wait()
        pltpu.make_async_copy(taby_hbm.at[yi.at[pl.ds(c * _C, _C)]], yr,
                              semg).wait()
        if c >= 2:
            pltpu.make_async_copy(
                toks[p], out_hbm.at[pl.ds(base + (c - 2) * _C, _C),
                                    pl.ds(0, 2 * Dq)], semos[p]).wait()
        if c + 1 < nch:
            start_tok_in(c + 1)

        def addrow(r, carry):
            for j in range(Dq // 16):
                s = pl.ds(j * 16, 16)
                s2 = pl.ds(Dq + j * 16, 16)
                tokb[r, s] = tokb[r, s] + xr[r, s]
                tokb[r, s2] = tokb[r, s2] + yr[r, s]
            return carry

        lax.fori_loop(0, _C, addrow, 0)
        if c + 1 < nch:
            start_gathers(c + 1)
        pltpu.make_async_copy(
            tokb, out_hbm.at[pl.ds(off, _C), pl.ds(0, 2 * Dq)],
            semos[p]).start()
    for c in (nch - 2, nch - 1):
        pltpu.make_async_copy(
            toks[c % 2],
            out_hbm.at[pl.ds(base + c * _C, _C), pl.ds(0, 2 * Dq)],
            semos[c % 2]).wait()


def _sc_spatial(tok, px, py, tabx, taby):
    BN, D = tok.shape
    Dq = tabx.shape[1]
    rows_w = BN // _NW
    mesh = plsc.VectorSubcoreMesh(core_axis_name="c", subcore_axis_name="s",
                                  num_cores=_NC, num_subcores=_NS)
    f = pl.kernel(
        _sc_body,
        out_type=jax.ShapeDtypeStruct((BN, D), jnp.float32),
        mesh=mesh,
        scratch_types=[
            pltpu.VMEM((rows_w,), jnp.float32),
            pltpu.VMEM((rows_w,), jnp.float32),
            pltpu.VMEM((rows_w,), jnp.int32),
            pltpu.VMEM((rows_w,), jnp.int32),
            pltpu.VMEM((_C, 2 * Dq), jnp.float32),
            pltpu.VMEM((_C, 2 * Dq), jnp.float32),
            pltpu.VMEM((_C, Dq), jnp.float32),
            pltpu.VMEM((_C, Dq), jnp.float32),
            pltpu.SemaphoreType.DMA,
            pltpu.SemaphoreType.DMA,
            pltpu.SemaphoreType.DMA,
            pltpu.SemaphoreType.DMA,
            pltpu.SemaphoreType.DMA,
        ],
    )
    return f(tok, px, py, tabx, taby)


def _tc_body(buf_ref, tok_ref, tp_ref, w1_ref, b1_ref, w2_ref, b2_ref,
             out_ref):
    del buf_ref
    t = tp_ref[...]
    h = t * w1_ref[...] + b1_ref[...]
    h = 0.5 * h * (1.0 + jax.lax.erf(h * 0.7071067811865476))
    temp = jnp.dot(h.astype(jnp.bfloat16), w2_ref[...].astype(jnp.bfloat16),
                   preferred_element_type=jnp.float32) + b2_ref[...]
    out_ref[...] = tok_ref[...] + temp


def _tc_temporal(buf, tok, tp, W1, b1r, W2, b2r):
    BN, D = tok.shape
    H = W1.shape[1]
    RB = 1024
    grid = (BN // RB,)
    rep = lambda i: (0, 0)
    return pl.pallas_call(
        _tc_body,
        grid=grid,
        in_specs=[
            pl.BlockSpec(memory_space=pl.ANY),
            pl.BlockSpec((RB, H), lambda i: (i, 1)),
            pl.BlockSpec((RB, 1), lambda i: (i, 0)),
            pl.BlockSpec((1, H), rep),
            pl.BlockSpec((1, H), rep),
            pl.BlockSpec((H, H), rep),
            pl.BlockSpec((1, H), rep),
        ],
        out_specs=pl.BlockSpec((RB, H), lambda i: (i, 1)),
        out_shape=jax.ShapeDtypeStruct((BN, D), jnp.float32),
        input_output_aliases={0: 0},
        compiler_params=pltpu.CompilerParams(
            dimension_semantics=("arbitrary",),
        ),
    )(buf, tok, tp, W1, b1r, W2, b2r)


@jax.jit
def kernel(tokens, spatial_positions, temporal_positions, spatial_embed_x,
           spatial_embed_y, W1, b1, W2, b2):
    B, N, D = tokens.shape
    BN = B * N
    R = spatial_embed_x.shape[1]
    H = W1.shape[1]

    tok = tokens.reshape(BN, D)
    sp = spatial_positions.reshape(BN, 2)
    px = sp[:, 0]
    py = sp[:, 1]
    tp = temporal_positions.reshape(BN, 1)
    tabx = spatial_embed_x.reshape(R, D // 4)
    taby = spatial_embed_y.reshape(R, D // 4)

    buf = _sc_spatial(tok, px, py, tabx, taby)
    out = _tc_temporal(buf, tok, tp, W1, b1.reshape(1, H), W2,
                       b2.reshape(1, H))
    return out.reshape(B, N, D)


# speedup vs baseline: 1.5722x; 1.2986x over previous
import functools

import jax
import jax.numpy as jnp
from jax import lax
from jax.experimental import pallas as pl
from jax.experimental.pallas import tpu as pltpu
from jax.experimental.pallas import tpu_sc as plsc

_NC = 2
_NS = 16
_NW = _NC * _NS
_C = 64


def _sc_body(tok_hbm, px_hbm, py_hbm, tabx_hbm, taby_hbm, out_hbm,
             pxv, pyv, xi, yi, iden,
             tokx0, tokx1, toky0, toky1, xr, yr,
             semi0, semi1, semo0, semo1, semg):
    R = tabx_hbm.shape[0]
    rows_w = tok_hbm.shape[0] // _NW
    nch = rows_w // _C
    Dq = tabx_hbm.shape[1]
    wid = lax.axis_index("s") * _NC + lax.axis_index("c")
    base = wid * rows_w

    pltpu.sync_copy(px_hbm.at[pl.ds(base, rows_w)], pxv)
    pltpu.sync_copy(py_hbm.at[pl.ds(base, rows_w)], pyv)
    for j in range(rows_w // 16):
        s = pl.ds(j * 16, 16)
        xi[s] = (pxv[s] * float(R)).astype(jnp.int32)
        yi[s] = (pyv[s] * float(R)).astype(jnp.int32)
    for j in range(_C // 16):
        iden[pl.ds(j * 16, 16)] = lax.iota(jnp.int32, 16) + (j * 16)

    tokxs = (tokx0, tokx1)
    tokys = (toky0, toky1)
    semis = (semi0, semi1)
    semos = (semo0, semo1)

    def in_copies(c):
        p = c % 2
        off = base + c * _C
        return (
            pltpu.make_async_copy(
                tok_hbm.at[pl.ds(off, _C), pl.ds(0, Dq)], tokxs[p],
                semis[p]),
            pltpu.make_async_copy(
                tok_hbm.at[pl.ds(off, _C), pl.ds(Dq, Dq)], tokys[p],
                semis[p]),
        )

    def gather_copies(c):
        return (
            pltpu.make_async_copy(
                tabx_hbm.at[xi.at[pl.ds(c * _C, _C)]], xr, semg),
            pltpu.make_async_copy(
                taby_hbm.at[yi.at[pl.ds(c * _C, _C)]], yr, semg),
        )

    def out_copies(c):
        p = c % 2
        off = base + c * _C
        return (
            pltpu.make_async_copy(
                tokxs[p], out_hbm.at[pl.ds(off, _C), pl.ds(0, Dq)],
                semos[p]),
            pltpu.make_async_copy(
                tokys[p], out_hbm.at[pl.ds(off, _C), pl.ds(Dq, Dq)],
                semos[p]),
        )

    for cp in in_copies(0) + gather_copies(0):
        cp.start()
    for c in range(nch):
        p = c % 2
        for cp in in_copies(c) + gather_copies(c):
            cp.wait()
        if c >= 2:
            for cp in out_copies(c - 2):
                cp.wait()
        if c + 1 < nch:
            for cp in in_copies(c + 1):
                cp.start()
        tokx, toky = tokxs[p], tokys[p]

        @plsc.parallel_loop(0, _C, unroll=2)
        def _addrow(r):
            for j in range(Dq // 16):
                s = pl.ds(j * 16, 16)
                tokx[r, s] = tokx[r, s] + xr[r, s]
                toky[r, s] = toky[r, s] + yr[r, s]

        if c + 1 < nch:
            for cp in gather_copies(c + 1):
                cp.start()
        for cp in out_copies(c):
            cp.start()
    for c in (nch - 2, nch - 1):
        for cp in out_copies(c):
            cp.wait()


def _sc_spatial(tok, px, py, tabx, taby):
    BN, D = tok.shape
    Dq = tabx.shape[1]
    rows_w = BN // _NW
    mesh = plsc.VectorSubcoreMesh(core_axis_name="c", subcore_axis_name="s",
                                  num_cores=_NC, num_subcores=_NS)
    f = pl.kernel(
        _sc_body,
        out_type=jax.ShapeDtypeStruct((BN, D), jnp.float32),
        mesh=mesh,
        scratch_types=[
            pltpu.VMEM((rows_w,), jnp.float32),
            pltpu.VMEM((rows_w,), jnp.float32),
            pltpu.VMEM((rows_w,), jnp.int32),
            pltpu.VMEM((rows_w,), jnp.int32),
            pltpu.VMEM((_C,), jnp.int32),
            pltpu.VMEM((_C, Dq), jnp.float32),
            pltpu.VMEM((_C, Dq), jnp.float32),
            pltpu.VMEM((_C, Dq), jnp.float32),
            pltpu.VMEM((_C, Dq), jnp.float32),
            pltpu.VMEM((_C, Dq), jnp.float32),
            pltpu.VMEM((_C, Dq), jnp.float32),
            pltpu.SemaphoreType.DMA,
            pltpu.SemaphoreType.DMA,
            pltpu.SemaphoreType.DMA,
            pltpu.SemaphoreType.DMA,
            pltpu.SemaphoreType.DMA,
        ],
    )
    return f(tok, px, py, tabx, taby)


def _tc_body(buf_ref, tok_ref, tp_ref, w1_ref, b1_ref, w2_ref, b2_ref,
             out_ref):
    del buf_ref
    t = tp_ref[...]
    h = t * w1_ref[...] + b1_ref[...]
    h = 0.5 * h * (1.0 + jax.lax.erf(h * 0.7071067811865476))
    temp = jnp.dot(h.astype(jnp.bfloat16), w2_ref[...].astype(jnp.bfloat16),
                   preferred_element_type=jnp.float32) + b2_ref[...]
    out_ref[...] = tok_ref[...] + temp


def _tc_temporal(buf, tok, tp, W1, b1r, W2, b2r):
    BN, D = tok.shape
    H = W1.shape[1]
    RB = 1024
    grid = (BN // RB,)
    rep = lambda i: (0, 0)
    return pl.pallas_call(
        _tc_body,
        grid=grid,
        in_specs=[
            pl.BlockSpec(memory_space=pl.ANY),
            pl.BlockSpec((RB, H), lambda i: (i, 1)),
            pl.BlockSpec((RB, 1), lambda i: (i, 0)),
            pl.BlockSpec((1, H), rep),
            pl.BlockSpec((1, H), rep),
            pl.BlockSpec((H, H), rep),
            pl.BlockSpec((1, H), rep),
        ],
        out_specs=pl.BlockSpec((RB, H), lambda i: (i, 1)),
        out_shape=jax.ShapeDtypeStruct((BN, D), jnp.float32),
        input_output_aliases={0: 0},
        compiler_params=pltpu.CompilerParams(
            dimension_semantics=("arbitrary",),
        ),
    )(buf, tok, tp, W1, b1r, W2, b2r)


@jax.jit
def kernel(tokens, spatial_positions, temporal_positions, spatial_embed_x,
           spatial_embed_y, W1, b1, W2, b2):
    B, N, D = tokens.shape
    BN = B * N
    R = spatial_embed_x.shape[1]
    H = W1.shape[1]

    tok = tokens.reshape(BN, D)
    sp = spatial_positions.reshape(BN, 2)
    px = sp[:, 0]
    py = sp[:, 1]
    tp = temporal_positions.reshape(BN, 1)
    tabx = spatial_embed_x.reshape(R, D // 4)
    taby = spatial_embed_y.reshape(R, D // 4)

    buf = _sc_spatial(tok, px, py, tabx, taby)
    out = _tc_temporal(buf, tok, tp, W1, b1.reshape(1, H), W2,
                       b2.reshape(1, H))
    return out.reshape(B, N, D)
